# baseline (speedup 1.0000x reference)
import functools

import jax
import jax.numpy as jnp
from jax import lax
from jax.experimental import pallas as pl
from jax.experimental.pallas import tpu as pltpu
from jax.experimental.pallas import tpu_sc as plsc

VOCAB = 100000
EMB = 32
CTX = 4
IN_DIM = CTX * EMB
HIDDEN = 128
BATCH = 1024
NLOOK = BATCH * CTX
LINES = VOCAB * EMB // 128


def _sc_gather(table_lines, idx_flat):
    info = plsc.get_sparse_core_info()
    nc, ns, nl = info.num_cores, info.num_subcores, info.num_lanes
    nw = nc * ns
    b_per_w = NLOOK // nw

    mesh = plsc.VectorSubcoreMesh(core_axis_name="c", subcore_axis_name="s")

    @functools.partial(
        pl.kernel,
        mesh=mesh,
        out_type=jax.ShapeDtypeStruct((NLOOK, 128), jnp.float32),
        scratch_types=[
            pltpu.VMEM((b_per_w,), jnp.int32),
            pltpu.VMEM((b_per_w,), jnp.int32),
            pltpu.VMEM((b_per_w, 128), jnp.float32),
            pltpu.SemaphoreType.DMA,
        ],
    )
    def gather_k(table_hbm, idx_hbm, out_hbm, idx_v, hi_v, rows_v, sem):
        wid = lax.axis_index("s") * nc + lax.axis_index("c")
        base = wid * b_per_w
        pltpu.sync_copy(idx_hbm.at[pl.ds(base, b_per_w)], idx_v)
        for i in range(b_per_w // nl):
            sl = pl.ds(i * nl, nl)
            hi_v[sl] = lax.shift_right_logical(idx_v[sl], 2)
        pltpu.async_copy(table_hbm.at[hi_v], rows_v, sem).wait()
        pltpu.sync_copy(rows_v, out_hbm.at[pl.ds(base, b_per_w)])

    return gather_k(table_lines, idx_flat)


def _hid_body(lines_ref, off_ref, hw_ref, hb_ref, feat_ref, hid_ref):
    parts = []
    for c in range(CTX):
        chunk = lines_ref[:, c * 128:(c + 1) * 128]
        off_c = off_ref[:, c:c + 1]
        sub = jnp.zeros((BATCH, EMB), jnp.float32)
        for k in range(4):
            cand = chunk[:, k * EMB:(k + 1) * EMB]
            sub = jnp.where(off_c == k, cand, sub)
        parts.append(sub)
    feat = jnp.concatenate(parts, axis=1)
    feat_ref[...] = feat.astype(jnp.bfloat16)
    acc = jnp.dot(feat, hw_ref[...], preferred_element_type=jnp.float32)
    hid_ref[...] = jnp.tanh(acc + hb_ref[...]).astype(jnp.bfloat16)


def _hidden(lines, off, H_w, H_b2):
    return pl.pallas_call(
        _hid_body,
        out_shape=(
            jax.ShapeDtypeStruct((BATCH, IN_DIM), jnp.bfloat16),
            jax.ShapeDtypeStruct((BATCH, HIDDEN), jnp.bfloat16),
        ),
    )(lines, off, H_w, H_b2)


TN = 512
NSTEP = pl.cdiv(VOCAB, TN)
TAIL = VOCAB - (NSTEP - 1) * TN
NBUF = 16


def _out_body(feat_ref, hid_ref, uw_ref, ww_ref, ub_ref, b_ref, out_ref,
              obuf, tbuf, sems, tsem):
    j = pl.program_id(0)
    slot = lax.rem(j, NBUF)

    def _copy(s, step):
        return pltpu.make_async_copy(
            obuf.at[s],
            out_ref.at[:, pl.ds(step * TN, TN)],
            sems.at[s],
        )

    @pl.when(jnp.logical_and(j >= NBUF, j < NSTEP))
    def _():
        _copy(slot, j - NBUF).wait()

    uw = uw_ref[...].astype(jnp.bfloat16)
    ww = ww_ref[...].astype(jnp.bfloat16)
    acc = jnp.dot(hid_ref[...], uw, preferred_element_type=jnp.float32)
    acc += jnp.dot(feat_ref[...], ww, preferred_element_type=jnp.float32)
    res = acc + ub_ref[...] + b_ref[...]

    @pl.when(j < NSTEP - 1)
    def _():
        obuf[slot] = res
        _copy(slot, j).start()

    @pl.when(j == NSTEP - 1)
    def _():
        tbuf[...] = res[:, :TAIL]
        pltpu.make_async_copy(
            tbuf, out_ref.at[:, pl.ds((NSTEP - 1) * TN, TAIL)], tsem,
        ).start()
        for step in range(NSTEP - NBUF, NSTEP - 1):
            _copy(step % NBUF, step).wait()
        pltpu.make_async_copy(
            tbuf, out_ref.at[:, pl.ds((NSTEP - 1) * TN, TAIL)], tsem,
        ).wait()


def _project(feat, hid, U_w, W_w, U_b2, b2):
    grid = (NSTEP,)
    return pl.pallas_call(
        _out_body,
        grid=grid,
        in_specs=[
            pl.BlockSpec((BATCH, IN_DIM), lambda j: (0, 0)),
            pl.BlockSpec((BATCH, HIDDEN), lambda j: (0, 0)),
            pl.BlockSpec((HIDDEN, TN), lambda j: (0, j)),
            pl.BlockSpec((IN_DIM, TN), lambda j: (0, j)),
            pl.BlockSpec((1, TN), lambda j: (0, j)),
            pl.BlockSpec((1, TN), lambda j: (0, j)),
        ],
        out_specs=pl.BlockSpec(memory_space=pltpu.MemorySpace.HBM),
        out_shape=jax.ShapeDtypeStruct((BATCH, VOCAB), jnp.float32),
        scratch_shapes=[
            pltpu.VMEM((NBUF, BATCH, TN), jnp.float32),
            pltpu.VMEM((BATCH, TAIL), jnp.float32),
            pltpu.SemaphoreType.DMA((NBUF,)),
            pltpu.SemaphoreType.DMA,
        ],
    )(feat, hid, U_w, W_w, U_b2, b2)


def kernel(x, C, H_w, H_b, U_w, U_b, W_w, b):
    xi = x.astype(jnp.int32)
    idx_flat = xi.reshape(-1)
    off = (xi & 3)
    table_lines = C.reshape(LINES, 128)
    lines = _sc_gather(table_lines, idx_flat)
    lines1024 = lines.reshape(BATCH, CTX * 128)
    feat, hid = _hidden(lines1024, off, H_w, H_b.reshape(1, HIDDEN))
    return _project(feat, hid, U_w, W_w,
                    U_b.reshape(1, VOCAB), b.reshape(1, VOCAB))


# speedup vs baseline: 3.1589x; 3.1589x over previous
import functools

import jax
import jax.numpy as jnp
from jax import lax
from jax.experimental import pallas as pl
from jax.experimental.pallas import tpu as pltpu
from jax.experimental.pallas import tpu_sc as plsc

VOCAB = 100000
EMB = 32
CTX = 4
IN_DIM = CTX * EMB
HIDDEN = 128
BATCH = 1024
NLOOK = BATCH * CTX
LINES = VOCAB * EMB // 128


def _sc_gather(table_lines, idx_flat):
    info = plsc.get_sparse_core_info()
    nc, ns, nl = info.num_cores, info.num_subcores, info.num_lanes
    nw = nc * ns
    b_per_w = NLOOK // nw

    mesh = plsc.VectorSubcoreMesh(core_axis_name="c", subcore_axis_name="s")

    @functools.partial(
        pl.kernel,
        mesh=mesh,
        out_type=jax.ShapeDtypeStruct((NLOOK, 128), jnp.float32),
        scratch_types=[
            pltpu.VMEM((b_per_w,), jnp.int32),
            pltpu.VMEM((b_per_w,), jnp.int32),
            pltpu.VMEM((b_per_w, 128), jnp.float32),
            pltpu.SemaphoreType.DMA,
        ],
    )
    def gather_k(table_hbm, idx_hbm, out_hbm, idx_v, hi_v, rows_v, sem):
        wid = lax.axis_index("s") * nc + lax.axis_index("c")
        base = wid * b_per_w
        pltpu.sync_copy(idx_hbm.at[pl.ds(base, b_per_w)], idx_v)
        for i in range(b_per_w // nl):
            sl = pl.ds(i * nl, nl)
            hi_v[sl] = lax.shift_right_logical(idx_v[sl], 2)
        pltpu.async_copy(table_hbm.at[hi_v], rows_v, sem).wait()
        pltpu.sync_copy(rows_v, out_hbm.at[pl.ds(base, b_per_w)])

    return gather_k(table_lines, idx_flat)


def _hid_body(lines_ref, off_ref, hw_ref, hb_ref, feat_ref, hid_ref):
    parts = []
    for c in range(CTX):
        chunk = lines_ref[:, c * 128:(c + 1) * 128]
        off_c = off_ref[:, c:c + 1]
        sub = jnp.zeros((BATCH, EMB), jnp.float32)
        for k in range(4):
            cand = chunk[:, k * EMB:(k + 1) * EMB]
            sub = jnp.where(off_c == k, cand, sub)
        parts.append(sub)
    feat = jnp.concatenate(parts, axis=1)
    feat_ref[...] = feat.astype(jnp.bfloat16)
    acc = jnp.dot(feat, hw_ref[...], preferred_element_type=jnp.float32)
    hid_ref[...] = jnp.tanh(acc + hb_ref[...]).astype(jnp.bfloat16)


def _hidden(lines, off, H_w, H_b2):
    return pl.pallas_call(
        _hid_body,
        out_shape=(
            jax.ShapeDtypeStruct((BATCH, IN_DIM), jnp.bfloat16),
            jax.ShapeDtypeStruct((BATCH, HIDDEN), jnp.bfloat16),
        ),
    )(lines, off, H_w, H_b2)


TN = 2048
NSTEP = pl.cdiv(VOCAB, TN)
TAIL = VOCAB - (NSTEP - 1) * TN
NBUF = 4


def _out_body(featT_ref, hidT_ref, uwT_ref, wwT_ref, ub_ref, b_ref, outT_ref,
              obuf, sems):
    j = pl.program_id(0)
    slot = lax.rem(j, NBUF)

    def _copy(s, step, height):
        return pltpu.make_async_copy(
            obuf.at[s, pl.ds(0, height)],
            outT_ref.at[pl.ds(step * TN, height)],
            sems.at[s],
        )

    @pl.when(j >= NBUF)
    def _():
        _copy(slot, j - NBUF, TN).wait()

    uwT = uwT_ref[...].astype(jnp.bfloat16)
    wwT = wwT_ref[...].astype(jnp.bfloat16)
    acc = jnp.dot(uwT, hidT_ref[...], preferred_element_type=jnp.float32)
    acc += jnp.dot(wwT, featT_ref[...], preferred_element_type=jnp.float32)
    bias = (ub_ref[...] + b_ref[...]).reshape(TN, 1)
    obuf[slot] = acc + bias

    @pl.when(j < NSTEP - 1)
    def _():
        _copy(slot, j, TN).start()

    @pl.when(j == NSTEP - 1)
    def _():
        _copy(slot, j, TAIL).start()
        for step in range(NSTEP - NBUF, NSTEP):
            height = TAIL if step == NSTEP - 1 else TN
            _copy(step % NBUF, step, height).wait()


def _project(featT, hidT, U_wT, W_wT, U_b, b):
    grid = (NSTEP,)
    return pl.pallas_call(
        _out_body,
        grid=grid,
        in_specs=[
            pl.BlockSpec((IN_DIM, BATCH), lambda j: (0, 0)),
            pl.BlockSpec((HIDDEN, BATCH), lambda j: (0, 0)),
            pl.BlockSpec((TN, HIDDEN), lambda j: (j, 0)),
            pl.BlockSpec((TN, IN_DIM), lambda j: (j, 0)),
            pl.BlockSpec((TN,), lambda j: (j,)),
            pl.BlockSpec((TN,), lambda j: (j,)),
        ],
        out_specs=pl.BlockSpec(memory_space=pltpu.MemorySpace.HBM),
        out_shape=jax.ShapeDtypeStruct((VOCAB, BATCH), jnp.float32),
        scratch_shapes=[
            pltpu.VMEM((NBUF, TN, BATCH), jnp.float32),
            pltpu.SemaphoreType.DMA((NBUF,)),
        ],
    )(featT, hidT, U_wT, W_wT, U_b, b)


def kernel(x, C, H_w, H_b, U_w, U_b, W_w, b):
    xi = x.astype(jnp.int32)
    idx_flat = xi.reshape(-1)
    off = (xi & 3)
    table_lines = C.reshape(LINES, 128)
    lines = _sc_gather(table_lines, idx_flat)
    lines1024 = lines.reshape(BATCH, CTX * 128)
    feat, hid = _hidden(lines1024, off, H_w, H_b.reshape(1, HIDDEN))
    outT = _project(feat.T, hid.T, U_w.T, W_w.T, U_b, b)
    return outT.T


# speedup vs baseline: 3.2868x; 1.0405x over previous
import functools

import jax
import jax.numpy as jnp
from jax import lax
from jax.experimental import pallas as pl
from jax.experimental.pallas import tpu as pltpu
from jax.experimental.pallas import tpu_sc as plsc

VOCAB = 100000
EMB = 32
CTX = 4
IN_DIM = CTX * EMB
HIDDEN = 128
BATCH = 1024
NLOOK = BATCH * CTX
LINES = VOCAB * EMB // 128


def _sc_gather(table, idx_flat):
    info = plsc.get_sparse_core_info()
    nc, ns = info.num_cores, info.num_subcores
    nw = nc * ns
    b_per_w = NLOOK // nw

    mesh = plsc.VectorSubcoreMesh(core_axis_name="c", subcore_axis_name="s")

    @functools.partial(
        pl.kernel,
        mesh=mesh,
        out_type=jax.ShapeDtypeStruct((NLOOK, EMB), jnp.float32),
        scratch_types=[
            pltpu.VMEM((b_per_w,), jnp.int32),
            pltpu.VMEM((b_per_w, EMB), jnp.float32),
            pltpu.SemaphoreType.DMA,
        ],
        compiler_params=pltpu.CompilerParams(use_tc_tiling_on_sc=False),
    )
    def gather_k(table_hbm, idx_hbm, out_hbm, idx_v, rows_v, sem):
        wid = lax.axis_index("s") * nc + lax.axis_index("c")
        base = wid * b_per_w
        pltpu.sync_copy(idx_hbm.at[pl.ds(base, b_per_w)], idx_v)
        pltpu.async_copy(table_hbm.at[idx_v], rows_v, sem).wait()
        pltpu.sync_copy(rows_v, out_hbm.at[pl.ds(base, b_per_w)])

    return gather_k(table, idx_flat)


def _hid_body(feat_ref, hw_ref, hb_ref, featb_ref, hid_ref):
    feat = feat_ref[...]
    featb_ref[...] = feat.astype(jnp.bfloat16)
    acc = jnp.dot(feat, hw_ref[...], preferred_element_type=jnp.float32)
    hid_ref[...] = jnp.tanh(acc + hb_ref[...]).astype(jnp.bfloat16)


def _hidden(feat, H_w, H_b2):
    return pl.pallas_call(
        _hid_body,
        out_shape=(
            jax.ShapeDtypeStruct((BATCH, IN_DIM), jnp.bfloat16),
            jax.ShapeDtypeStruct((BATCH, HIDDEN), jnp.bfloat16),
        ),
    )(feat, H_w, H_b2)


TN = 2048
NSTEP = pl.cdiv(VOCAB, TN)
TAIL = VOCAB - (NSTEP - 1) * TN
NBUF = 4


def _out_body(featT_ref, hidT_ref, uwT_ref, wwT_ref, ub_ref, b_ref, outT_ref,
              obuf, sems):
    j = pl.program_id(0)
    slot = lax.rem(j, NBUF)

    def _copy(s, step, height):
        return pltpu.make_async_copy(
            obuf.at[s, pl.ds(0, height)],
            outT_ref.at[pl.ds(step * TN, height)],
            sems.at[s],
        )

    @pl.when(j >= NBUF)
    def _():
        _copy(slot, j - NBUF, TN).wait()

    uwT = uwT_ref[...].astype(jnp.bfloat16)
    wwT = wwT_ref[...].astype(jnp.bfloat16)
    acc = jnp.dot(uwT, hidT_ref[...], preferred_element_type=jnp.float32)
    acc += jnp.dot(wwT, featT_ref[...], preferred_element_type=jnp.float32)
    bias = (ub_ref[...] + b_ref[...]).reshape(TN, 1)
    obuf[slot] = acc + bias

    @pl.when(j < NSTEP - 1)
    def _():
        _copy(slot, j, TN).start()

    @pl.when(j == NSTEP - 1)
    def _():
        _copy(slot, j, TAIL).start()
        for step in range(NSTEP - NBUF, NSTEP):
            height = TAIL if step == NSTEP - 1 else TN
            _copy(step % NBUF, step, height).wait()


def _project(featT, hidT, U_wT, W_wT, U_b, b):
    grid = (NSTEP,)
    return pl.pallas_call(
        _out_body,
        grid=grid,
        in_specs=[
            pl.BlockSpec((IN_DIM, BATCH), lambda j: (0, 0)),
            pl.BlockSpec((HIDDEN, BATCH), lambda j: (0, 0)),
            pl.BlockSpec((TN, HIDDEN), lambda j: (j, 0)),
            pl.BlockSpec((TN, IN_DIM), lambda j: (j, 0)),
            pl.BlockSpec((TN,), lambda j: (j,)),
            pl.BlockSpec((TN,), lambda j: (j,)),
        ],
        out_specs=pl.BlockSpec(memory_space=pltpu.MemorySpace.HBM),
        out_shape=jax.ShapeDtypeStruct((VOCAB, BATCH), jnp.float32),
        scratch_shapes=[
            pltpu.VMEM((NBUF, TN, BATCH), jnp.float32),
            pltpu.SemaphoreType.DMA((NBUF,)),
        ],
    )(featT, hidT, U_wT, W_wT, U_b, b)


def kernel(x, C, H_w, H_b, U_w, U_b, W_w, b):
    idx_flat = x.astype(jnp.int32).reshape(-1)
    emb = _sc_gather(C, idx_flat)
    feat0 = emb.reshape(BATCH, IN_DIM)
    feat, hid = _hidden(feat0, H_w, H_b.reshape(1, HIDDEN))
    outT = _project(feat.T, hid.T, U_w.T, W_w.T, U_b, b)
    return outT.T


# speedup vs baseline: 3.4235x; 1.0416x over previous
import functools

import jax
import jax.numpy as jnp
from jax import lax
from jax.experimental import pallas as pl
from jax.experimental.pallas import tpu as pltpu
from jax.experimental.pallas import tpu_sc as plsc

VOCAB = 100000
EMB = 32
CTX = 4
IN_DIM = CTX * EMB
HIDDEN = 128
BATCH = 1024
NLOOK = BATCH * CTX
LINES = VOCAB * EMB // 128


BL = 1024
NPACK = pl.cdiv(VOCAB, 4 * BL)
LINES_P = NPACK * BL
KMAX = (VOCAB - 1) // BL


def _pack_body(ct0_ref, ct1_ref, ct2_ref, ct3_ref, out_ref):
    for c, ref in enumerate((ct0_ref, ct1_ref, ct2_ref, ct3_ref)):
        out_ref[:, c * EMB:(c + 1) * EMB] = ref[...].T


def _pack(ct):
    specs = [
        pl.BlockSpec(
            (EMB, BL),
            (lambda c: (lambda j: (0, jnp.minimum(4 * j + c, KMAX))))(c))
        for c in range(4)
    ]
    return pl.pallas_call(
        _pack_body,
        grid=(NPACK,),
        in_specs=specs,
        out_specs=pl.BlockSpec((BL, 128), lambda j: (j, 0)),
        out_shape=jax.ShapeDtypeStruct((LINES_P, 128), jnp.float32),
    )(ct, ct, ct, ct)


def _sc_gather(table_lines, idx_flat):
    info = plsc.get_sparse_core_info()
    nc, ns, nl = info.num_cores, info.num_subcores, info.num_lanes
    nw = nc * ns
    b_per_w = NLOOK // nw

    mesh = plsc.VectorSubcoreMesh(core_axis_name="c", subcore_axis_name="s")

    @functools.partial(
        pl.kernel,
        mesh=mesh,
        out_type=jax.ShapeDtypeStruct((NLOOK, 128), jnp.float32),
        scratch_types=[
            pltpu.VMEM((b_per_w,), jnp.int32),
            pltpu.VMEM((b_per_w,), jnp.int32),
            pltpu.VMEM((b_per_w, 128), jnp.float32),
            pltpu.SemaphoreType.DMA,
        ],
    )
    def gather_k(table_hbm, idx_hbm, out_hbm, idx_v, hi_v, rows_v, sem):
        wid = lax.axis_index("s") * nc + lax.axis_index("c")
        base = wid * b_per_w
        pltpu.sync_copy(idx_hbm.at[pl.ds(base, b_per_w)], idx_v)
        for i in range(b_per_w // nl):
            sl = pl.ds(i * nl, nl)
            v = idx_v[sl]
            hi_v[sl] = (lax.shift_left(lax.shift_right_logical(v, 12), 10)
                        + (v & 1023))
        pltpu.async_copy(table_hbm.at[hi_v], rows_v, sem).wait()
        pltpu.sync_copy(rows_v, out_hbm.at[pl.ds(base, b_per_w)])

    return gather_k(table_lines, idx_flat)


def _hid_body(lines_ref, off_ref, hw_ref, hb_ref, feat_ref, hid_ref):
    parts = []
    for c in range(CTX):
        chunk = lines_ref[:, c * 128:(c + 1) * 128]
        off_c = off_ref[:, c:c + 1]
        sub = jnp.zeros((BATCH, EMB), jnp.float32)
        for k in range(4):
            cand = chunk[:, k * EMB:(k + 1) * EMB]
            sub = jnp.where(off_c == k, cand, sub)
        parts.append(sub)
    feat = jnp.concatenate(parts, axis=1)
    feat_ref[...] = feat.astype(jnp.bfloat16)
    acc = jnp.dot(feat, hw_ref[...], preferred_element_type=jnp.float32)
    hid_ref[...] = jnp.tanh(acc + hb_ref[...]).astype(jnp.bfloat16)


def _hidden(lines, off, H_w, H_b2):
    return pl.pallas_call(
        _hid_body,
        out_shape=(
            jax.ShapeDtypeStruct((BATCH, IN_DIM), jnp.bfloat16),
            jax.ShapeDtypeStruct((BATCH, HIDDEN), jnp.bfloat16),
        ),
    )(lines, off, H_w, H_b2)


TN = 2048
NSTEP = pl.cdiv(VOCAB, TN)
TAIL = VOCAB - (NSTEP - 1) * TN
NBUF = 4


def _out_body(featT_ref, hidT_ref, uwT_ref, wwT_ref, ub_ref, b_ref, outT_ref,
              obuf, sems):
    j = pl.program_id(0)
    slot = lax.rem(j, NBUF)

    def _copy(s, step, height):
        return pltpu.make_async_copy(
            obuf.at[s, pl.ds(0, height)],
            outT_ref.at[pl.ds(step * TN, height)],
            sems.at[s],
        )

    @pl.when(j >= NBUF)
    def _():
        _copy(slot, j - NBUF, TN).wait()

    uwT = uwT_ref[...].astype(jnp.bfloat16)
    wwT = wwT_ref[...].astype(jnp.bfloat16)
    acc = jnp.dot(uwT, hidT_ref[...], preferred_element_type=jnp.float32)
    acc += jnp.dot(wwT, featT_ref[...], preferred_element_type=jnp.float32)
    bias = (ub_ref[...] + b_ref[...]).reshape(TN, 1)
    obuf[slot] = acc + bias

    @pl.when(j < NSTEP - 1)
    def _():
        _copy(slot, j, TN).start()

    @pl.when(j == NSTEP - 1)
    def _():
        _copy(slot, j, TAIL).start()
        for step in range(NSTEP - NBUF, NSTEP):
            height = TAIL if step == NSTEP - 1 else TN
            _copy(step % NBUF, step, height).wait()


def _project(featT, hidT, U_wT, W_wT, U_b, b):
    grid = (NSTEP,)
    return pl.pallas_call(
        _out_body,
        grid=grid,
        in_specs=[
            pl.BlockSpec((IN_DIM, BATCH), lambda j: (0, 0)),
            pl.BlockSpec((HIDDEN, BATCH), lambda j: (0, 0)),
            pl.BlockSpec((TN, HIDDEN), lambda j: (j, 0)),
            pl.BlockSpec((TN, IN_DIM), lambda j: (j, 0)),
            pl.BlockSpec((TN,), lambda j: (j,)),
            pl.BlockSpec((TN,), lambda j: (j,)),
        ],
        out_specs=pl.BlockSpec(memory_space=pltpu.MemorySpace.HBM),
        out_shape=jax.ShapeDtypeStruct((VOCAB, BATCH), jnp.float32),
        scratch_shapes=[
            pltpu.VMEM((NBUF, TN, BATCH), jnp.float32),
            pltpu.SemaphoreType.DMA((NBUF,)),
        ],
    )(featT, hidT, U_wT, W_wT, U_b, b)


def kernel(x, C, H_w, H_b, U_w, U_b, W_w, b):
    xi = x.astype(jnp.int32)
    idx_flat = xi.reshape(-1)
    off = (xi >> 10) & 3
    table_lines = _pack(C.T)
    lines = _sc_gather(table_lines, idx_flat)
    lines1024 = lines.reshape(BATCH, CTX * 128)
    feat, hid = _hidden(lines1024, off, H_w, H_b.reshape(1, HIDDEN))
    outT = _project(feat.T, hid.T, U_w.T, W_w.T, U_b, b)
    return outT.T
